# baseline (speedup 1.0000x reference)
import functools

import numpy as np
import jax
import jax.numpy as jnp
from jax import lax
from jax.experimental import pallas as pl
from jax.experimental.pallas import tpu as pltpu
from jax.experimental.pallas import tpu_sc as plsc

_FEATURE_DIMS = [3847] * 26
_F = 26
_D = 16
_TOTAL_ROWS = sum(_FEATURE_DIMS) + 1
_VPAD = 100096
_BATCH = 4096
_NPAIR = 325
_OUT_LEN = _NPAIR * _D

_NW = 32
_ROWS_PER_W = _BATCH // _NW

_TROWS = _F * _D
_GROWS = _VPAD * 4
_NCHUNK = _VPAD // 128
_TAIL_CI = _NCHUNK - 1
_TAIL_W = _TOTAL_ROWS - _TAIL_CI * 128

_i_idx, _j_idx = np.triu_indices(_F, k=1)
_OFFSETS = np.concatenate([[0], np.cumsum(_FEATURE_DIMS)[:-1]]).astype(np.int32)

_KDIV4 = np.minimum(np.arange(112) // 4, _F - 1).astype(np.int32)
_KMOD4 = (np.arange(112) % 4).astype(np.int32)

_RA = (_i_idx * 4 + _j_idx // 8).astype(np.int32)
_CA = ((_j_idx * 16) % 128).astype(np.int32)
_RB = (_j_idx * 4 + _i_idx // 8).astype(np.int32)
_CB = ((_i_idx * 16) % 128).astype(np.int32)


_K1RING = 8


def _k1_body(tab_hbm, tail_hbm, g_hbm, *refs):
    in_bufs = refs[0:_K1RING]
    out_blk = refs[_K1RING]
    isem = refs[_K1RING + 1]
    osem = refs[_K1RING + 2]
    cid = lax.axis_index("c")
    sid = lax.axis_index("s")
    wid = sid * 2 + cid
    iota16 = lax.broadcasted_iota(jnp.int32, (16,), 0)

    def fetch(src, t, buf, v0):
        pltpu.async_copy(src.at[pl.ds(t * _D, _D), pl.ds(v0, 128)], buf, isem)

    def wait_in(buf):
        pltpu.make_async_copy(tab_hbm.at[pl.ds(0, _D), pl.ds(0, 128)],
                              buf, isem).wait()

    def do_chunk(src, v0, ci):
        for t0 in range(_K1RING - 1):
            fetch(src, t0, in_bufs[t0], v0)
        for t in range(_F):
            buf = in_bufs[t % _K1RING]
            wait_in(buf)
            if t + _K1RING - 1 < _F:
                fetch(src, t + _K1RING - 1,
                      in_bufs[(t + _K1RING - 1) % _K1RING], v0)
            qt = (t * _D) // 128
            ct = (t * _D) % 128

            def v_body(vi, c2):
                v = vi * 4
                for u in range(4):
                    emb = plsc.load_gather(
                        buf, [iota16, jnp.broadcast_to(v + u, (16,))])
                    out_blk[(v + u) * 4 + qt, pl.ds(ct, _D)] = emb
                return c2

            lax.fori_loop(0, 32, v_body, 0)
        cpy = pltpu.make_async_copy(out_blk, g_hbm.at[pl.ds(ci * 512, 512), :],
                                    osem)
        cpy.start()
        cpy.wait()

    def outer(i, carry):
        ci = i * _NW + wid
        do_chunk(tab_hbm, ci * 128, ci)
        return carry

    nfull = (_NCHUNK - 1) // _NW
    lax.fori_loop(0, nfull, outer, 0)
    rem = (_NCHUNK - 1) - nfull * _NW

    @pl.when(wid < rem)
    def _():
        ci = nfull * _NW + wid
        do_chunk(tab_hbm, ci * 128, ci)

    @pl.when(wid == rem)
    def _():
        do_chunk(tail_hbm, 0, _TAIL_CI)


def _k2_body(xa_hbm, kd4_hbm, km4_hbm, g_hbm, out_hbm, kd4_v, km4_v, *refs):
    xarow_b = refs[0:2]
    idx_b = refs[2:4]
    blk_b = refs[4:6]
    out_b = refs[6:8]
    gsem = refs[8:10]
    osem = refs[10:12]

    cid = lax.axis_index("c")
    sid = lax.axis_index("s")
    wid = sid * 2 + cid
    base = wid * _ROWS_PER_W

    pltpu.sync_copy(kd4_hbm, kd4_v)
    pltpu.sync_copy(km4_hbm, km4_v)

    def fire_row(r, b):
        pltpu.sync_copy(xa_hbm.at[base + r], xarow_b[b])
        idx_v = idx_b[b]

        def q_body(q, c2):
            cols = kd4_v[pl.ds(q * 16, 16)]
            vs = plsc.load_gather(xarow_b[b], [cols])
            idx_v[pl.ds(q * 16, 16)] = vs * 4 + km4_v[pl.ds(q * 16, 16)]
            return c2

        lax.fori_loop(0, 7, q_body, 0)
        pltpu.async_copy(g_hbm.at[idx_v], blk_b[b], gsem[b])

    for b in range(2):
        fire_row(b, b)

    def row_loop(go, carry):
        for b in range(2):
            r = go * 2 + b
            blk_v, out_v = blk_b[b], out_b[b]
            pltpu.make_async_copy(g_hbm.at[pl.ds(0, 112)], blk_v,
                                  gsem[b]).wait()

            @pl.when(r >= 2)
            def _():
                pltpu.make_async_copy(out_v, out_hbm.at[pl.ds(0, _OUT_LEN)],
                                      osem[b]).wait()

            for p in range(_NPAIR):
                a = blk_v[int(_RA[p]), pl.ds(int(_CA[p]), _D)]
                bb = blk_v[int(_RB[p]), pl.ds(int(_CB[p]), _D)]
                out_v[pl.ds(p * _D, _D)] = a * bb
            pltpu.async_copy(out_v,
                             out_hbm.at[pl.ds((base + r) * _OUT_LEN,
                                              _OUT_LEN)], osem[b])

            @pl.when(r + 2 < _ROWS_PER_W)
            def _():
                fire_row(r + 2, b)
        return carry

    lax.fori_loop(0, _ROWS_PER_W // 2, row_loop, 0)
    for b in range(2):
        pltpu.make_async_copy(out_b[b], out_hbm.at[pl.ds(0, _OUT_LEN)],
                              osem[b]).wait()


def _k3_body(mid_hbm, out_hbm, in_blk, obuf, isem, osem):
    cid = lax.axis_index("c")
    sid = lax.axis_index("s")
    wid = sid * 2 + cid
    b0 = wid * 128
    iota16 = lax.broadcasted_iota(jnp.int32, (16,), 0)

    def do_pchunk(pc, width):
        wf = width * _D
        for i in range(128):
            pltpu.async_copy(
                mid_hbm.at[pl.ds((b0 + i) * _OUT_LEN + pc * 128, wf)],
                in_blk.at[pl.ds(i * wf, wf)], isem)
        pltpu.make_async_copy(
            mid_hbm.at[pl.ds(0, 128 * wf)], in_blk.at[pl.ds(0, 128 * wf)],
            isem).wait()

        def b_body(b, c2):
            for k in range(width):
                emb = in_blk[pl.ds(b * wf + k * _D, _D)]
                plsc.store_scatter(obuf, [k * _D + iota16,
                                          jnp.broadcast_to(b, (16,))], emb)
            return c2

        lax.fori_loop(0, 128, b_body, 0)
        for k in range(width):
            pltpu.async_copy(obuf.at[pl.ds(k * _D, _D), :],
                             out_hbm.at[pl.ds((pc * 8 + k) * _D, _D),
                                        pl.ds(b0, 128)], osem)
        for k in range(width):
            pltpu.make_async_copy(obuf.at[pl.ds(k * _D, _D), :],
                                  out_hbm.at[pl.ds(0, _D), pl.ds(0, 128)],
                                  osem).wait()

    def outer(pc, carry):
        do_pchunk(pc, 8)
        return carry

    lax.fori_loop(0, 40, outer, 0)
    do_pchunk(40, 5)


@functools.cache
def _build_k3():
    mesh = plsc.VectorSubcoreMesh(core_axis_name="c", subcore_axis_name="s")
    return pl.kernel(
        _k3_body,
        mesh=mesh,
        compiler_params=pltpu.CompilerParams(
            needs_layout_passes=False, use_tc_tiling_on_sc=True),
        out_type=jax.ShapeDtypeStruct((_OUT_LEN, _BATCH), jnp.float32),
        scratch_types=[
            pltpu.VMEM((128 * 128,), jnp.float32),
            pltpu.VMEM((128, 128), jnp.float32),
            pltpu.SemaphoreType.DMA,
            pltpu.SemaphoreType.DMA,
        ],
    )


@functools.cache
def _build_k1():
    mesh = plsc.VectorSubcoreMesh(core_axis_name="c", subcore_axis_name="s")
    return pl.kernel(
        _k1_body,
        mesh=mesh,
        compiler_params=pltpu.CompilerParams(
            needs_layout_passes=False, use_tc_tiling_on_sc=True,
            has_side_effects=False),
        out_type=jax.ShapeDtypeStruct((_GROWS, 128), jnp.float32),
        scratch_types=(
            [pltpu.VMEM((_D, 128), jnp.float32) for _ in range(_K1RING)]
            + [
                pltpu.VMEM((512, 128), jnp.float32),
                pltpu.SemaphoreType.DMA,
                pltpu.SemaphoreType.DMA,
            ]
        ),
    )


@functools.cache
def _build_k2():
    mesh = plsc.VectorSubcoreMesh(core_axis_name="c", subcore_axis_name="s")
    scratch = [
        pltpu.VMEM((112,), jnp.int32),
        pltpu.VMEM((112,), jnp.int32),
    ]
    scratch += [pltpu.VMEM((_F,), jnp.int32) for _ in range(2)]
    scratch += [pltpu.VMEM((112,), jnp.int32) for _ in range(2)]
    scratch += [pltpu.VMEM((112, 128), jnp.float32) for _ in range(2)]
    scratch += [pltpu.VMEM((_OUT_LEN,), jnp.float32) for _ in range(2)]
    scratch += [pltpu.SemaphoreType.DMA for _ in range(4)]
    return pl.kernel(
        _k2_body,
        mesh=mesh,
        compiler_params=pltpu.CompilerParams(
            needs_layout_passes=False, use_tc_tiling_on_sc=True),
        out_type=jax.ShapeDtypeStruct((_BATCH * _OUT_LEN,), jnp.float32),
        scratch_types=scratch,
    )


@jax.jit
def kernel(x, tables):
    xa = x + jnp.asarray(_OFFSETS)[None, :]
    tab2 = tables.transpose(0, 2, 1).reshape(_TROWS, _TOTAL_ROWS)
    tail = jnp.pad(tab2[:, _TAIL_CI * 128:], ((0, 0), (0, 128 - _TAIL_W)))
    g = _build_k1()(tab2, tail)
    mid = _build_k2()(xa, jnp.asarray(_KDIV4), jnp.asarray(_KMOD4), g)
    out2d = _build_k3()(mid)
    return out2d.reshape(_NPAIR, _D, _BATCH).transpose(2, 0, 1)


# speedup vs baseline: 1.3144x; 1.3144x over previous
import functools

import numpy as np
import jax
import jax.numpy as jnp
from jax import lax
from jax.experimental import pallas as pl
from jax.experimental.pallas import tpu as pltpu
from jax.experimental.pallas import tpu_sc as plsc

_FEATURE_DIMS = [3847] * 26
_F = 26
_D = 16
_TOTAL_ROWS = sum(_FEATURE_DIMS) + 1
_VPAD = 100096
_BATCH = 4096
_NPAIR = 325
_OUT_LEN = _NPAIR * _D

_NW = 32
_ROWS_PER_W = _BATCH // _NW

_TROWS = _F * _D
_GROWS = _VPAD * 4
_NCHUNK = _VPAD // 128
_TAIL_CI = _NCHUNK - 1
_TAIL_W = _TOTAL_ROWS - _TAIL_CI * 128

_i_idx, _j_idx = np.triu_indices(_F, k=1)
_OFFSETS = np.concatenate([[0], np.cumsum(_FEATURE_DIMS)[:-1]]).astype(np.int32)

_KDIV4 = np.minimum(np.arange(112) // 4, _F - 1).astype(np.int32)
_KMOD4 = (np.arange(112) % 4).astype(np.int32)

_RA = (_i_idx * 4 + _j_idx // 8).astype(np.int32)
_CA = ((_j_idx * 16) % 128).astype(np.int32)
_RB = (_j_idx * 4 + _i_idx // 8).astype(np.int32)
_CB = ((_i_idx * 16) % 128).astype(np.int32)


_K1RING = 8


def _k1_body(tab_hbm, tail_hbm, g_hbm, *refs):
    in_bufs = refs[0:_K1RING]
    out_blk = refs[_K1RING]
    isem = refs[_K1RING + 1]
    osem = refs[_K1RING + 2]
    cid = lax.axis_index("c")
    sid = lax.axis_index("s")
    wid = sid * 2 + cid
    iota16 = lax.broadcasted_iota(jnp.int32, (16,), 0)

    def fetch(src, t, buf, v0):
        pltpu.async_copy(src.at[pl.ds(t * _D, _D), pl.ds(v0, 128)], buf, isem)

    def wait_in(buf):
        pltpu.make_async_copy(tab_hbm.at[pl.ds(0, _D), pl.ds(0, 128)],
                              buf, isem).wait()

    def do_chunk(src, v0, ci):
        for t0 in range(_K1RING - 1):
            fetch(src, t0, in_bufs[t0], v0)
        for t in range(_F):
            buf = in_bufs[t % _K1RING]
            wait_in(buf)
            if t + _K1RING - 1 < _F:
                fetch(src, t + _K1RING - 1,
                      in_bufs[(t + _K1RING - 1) % _K1RING], v0)
            qt = (t * _D) // 128
            ct = (t * _D) % 128

            @plsc.parallel_loop(0, 128, 1, unroll=8)
            def _(v):
                emb = plsc.load_gather(
                    buf, [iota16, jnp.broadcast_to(v, (16,))])
                out_blk[v * 4 + qt, pl.ds(ct, _D)] = emb
        cpy = pltpu.make_async_copy(out_blk, g_hbm.at[pl.ds(ci * 512, 512), :],
                                    osem)
        cpy.start()
        cpy.wait()

    def outer(i, carry):
        ci = i * _NW + wid
        do_chunk(tab_hbm, ci * 128, ci)
        return carry

    nfull = (_NCHUNK - 1) // _NW
    lax.fori_loop(0, nfull, outer, 0)
    rem = (_NCHUNK - 1) - nfull * _NW

    @pl.when(wid < rem)
    def _():
        ci = nfull * _NW + wid
        do_chunk(tab_hbm, ci * 128, ci)

    @pl.when(wid == rem)
    def _():
        do_chunk(tail_hbm, 0, _TAIL_CI)


def _k2_body(xa_hbm, kd4_hbm, km4_hbm, g_hbm, out_hbm, kd4_v, km4_v, *refs):
    xarow_b = refs[0:2]
    idx_b = refs[2:4]
    blk_b = refs[4:6]
    out_b = refs[6:8]
    gsem = refs[8:10]
    osem = refs[10:12]

    cid = lax.axis_index("c")
    sid = lax.axis_index("s")
    wid = sid * 2 + cid
    base = wid * _ROWS_PER_W

    pltpu.sync_copy(kd4_hbm, kd4_v)
    pltpu.sync_copy(km4_hbm, km4_v)

    def fire_row(r, b):
        pltpu.sync_copy(xa_hbm.at[base + r], xarow_b[b])
        idx_v = idx_b[b]

        def q_body(q, c2):
            cols = kd4_v[pl.ds(q * 16, 16)]
            vs = plsc.load_gather(xarow_b[b], [cols])
            idx_v[pl.ds(q * 16, 16)] = vs * 4 + km4_v[pl.ds(q * 16, 16)]
            return c2

        lax.fori_loop(0, 7, q_body, 0)
        pltpu.async_copy(g_hbm.at[idx_v], blk_b[b], gsem[b])

    for b in range(2):
        fire_row(b, b)

    def row_loop(go, carry):
        for b in range(2):
            r = go * 2 + b
            blk_v, out_v = blk_b[b], out_b[b]
            pltpu.make_async_copy(g_hbm.at[pl.ds(0, 112)], blk_v,
                                  gsem[b]).wait()

            @pl.when(r >= 2)
            def _():
                pltpu.make_async_copy(out_v, out_hbm.at[pl.ds(0, _OUT_LEN)],
                                      osem[b]).wait()

            for p in range(_NPAIR):
                a = blk_v[int(_RA[p]), pl.ds(int(_CA[p]), _D)]
                bb = blk_v[int(_RB[p]), pl.ds(int(_CB[p]), _D)]
                out_v[pl.ds(p * _D, _D)] = a * bb
            pltpu.async_copy(out_v,
                             out_hbm.at[pl.ds((base + r) * _OUT_LEN,
                                              _OUT_LEN)], osem[b])

            @pl.when(r + 2 < _ROWS_PER_W)
            def _():
                fire_row(r + 2, b)
        return carry

    lax.fori_loop(0, _ROWS_PER_W // 2, row_loop, 0)
    for b in range(2):
        pltpu.make_async_copy(out_b[b], out_hbm.at[pl.ds(0, _OUT_LEN)],
                              osem[b]).wait()


def _k3_body(mid_hbm, out_hbm, in_blk, obuf, isem, osem):
    cid = lax.axis_index("c")
    sid = lax.axis_index("s")
    wid = sid * 2 + cid
    b0 = wid * 128
    iota16 = lax.broadcasted_iota(jnp.int32, (16,), 0)

    def do_pchunk(pc, width):
        wf = width * _D
        for i in range(128):
            pltpu.async_copy(
                mid_hbm.at[pl.ds((b0 + i) * _OUT_LEN + pc * 128, wf)],
                in_blk.at[pl.ds(i * wf, wf)], isem)
        pltpu.make_async_copy(
            mid_hbm.at[pl.ds(0, 128 * wf)], in_blk.at[pl.ds(0, 128 * wf)],
            isem).wait()

        def b_body(b, c2):
            for k in range(width):
                emb = in_blk[pl.ds(b * wf + k * _D, _D)]
                plsc.store_scatter(obuf, [k * _D + iota16,
                                          jnp.broadcast_to(b, (16,))], emb)
            return c2

        lax.fori_loop(0, 128, b_body, 0)
        for k in range(width):
            pltpu.async_copy(obuf.at[pl.ds(k * _D, _D), :],
                             out_hbm.at[pl.ds((pc * 8 + k) * _D, _D),
                                        pl.ds(b0, 128)], osem)
        for k in range(width):
            pltpu.make_async_copy(obuf.at[pl.ds(k * _D, _D), :],
                                  out_hbm.at[pl.ds(0, _D), pl.ds(0, 128)],
                                  osem).wait()

    def outer(pc, carry):
        do_pchunk(pc, 8)
        return carry

    lax.fori_loop(0, 40, outer, 0)
    do_pchunk(40, 5)


@functools.cache
def _build_k3():
    mesh = plsc.VectorSubcoreMesh(core_axis_name="c", subcore_axis_name="s")
    return pl.kernel(
        _k3_body,
        mesh=mesh,
        compiler_params=pltpu.CompilerParams(
            needs_layout_passes=False, use_tc_tiling_on_sc=True),
        out_type=jax.ShapeDtypeStruct((_OUT_LEN, _BATCH), jnp.float32),
        scratch_types=[
            pltpu.VMEM((128 * 128,), jnp.float32),
            pltpu.VMEM((128, 128), jnp.float32),
            pltpu.SemaphoreType.DMA,
            pltpu.SemaphoreType.DMA,
        ],
    )


@functools.cache
def _build_k1():
    mesh = plsc.VectorSubcoreMesh(core_axis_name="c", subcore_axis_name="s")
    return pl.kernel(
        _k1_body,
        mesh=mesh,
        compiler_params=pltpu.CompilerParams(
            needs_layout_passes=False, use_tc_tiling_on_sc=True,
            has_side_effects=False),
        out_type=jax.ShapeDtypeStruct((_GROWS, 128), jnp.float32),
        scratch_types=(
            [pltpu.VMEM((_D, 128), jnp.float32) for _ in range(_K1RING)]
            + [
                pltpu.VMEM((512, 128), jnp.float32),
                pltpu.SemaphoreType.DMA,
                pltpu.SemaphoreType.DMA,
            ]
        ),
    )


@functools.cache
def _build_k2():
    mesh = plsc.VectorSubcoreMesh(core_axis_name="c", subcore_axis_name="s")
    scratch = [
        pltpu.VMEM((112,), jnp.int32),
        pltpu.VMEM((112,), jnp.int32),
    ]
    scratch += [pltpu.VMEM((_F,), jnp.int32) for _ in range(2)]
    scratch += [pltpu.VMEM((112,), jnp.int32) for _ in range(2)]
    scratch += [pltpu.VMEM((112, 128), jnp.float32) for _ in range(2)]
    scratch += [pltpu.VMEM((_OUT_LEN,), jnp.float32) for _ in range(2)]
    scratch += [pltpu.SemaphoreType.DMA for _ in range(4)]
    return pl.kernel(
        _k2_body,
        mesh=mesh,
        compiler_params=pltpu.CompilerParams(
            needs_layout_passes=False, use_tc_tiling_on_sc=True),
        out_type=jax.ShapeDtypeStruct((_BATCH * _OUT_LEN,), jnp.float32),
        scratch_types=scratch,
    )


@jax.jit
def kernel(x, tables):
    xa = x + jnp.asarray(_OFFSETS)[None, :]
    tab2 = tables.transpose(0, 2, 1).reshape(_TROWS, _TOTAL_ROWS)
    tail = jnp.pad(tab2[:, _TAIL_CI * 128:], ((0, 0), (0, 128 - _TAIL_W)))
    g = _build_k1()(tab2, tail)
    mid = _build_k2()(xa, jnp.asarray(_KDIV4), jnp.asarray(_KMOD4), g)
    out2d = _build_k3()(mid)
    return out2d.reshape(_NPAIR, _D, _BATCH).transpose(2, 0, 1)
